# Initial kernel scaffold; baseline (speedup 1.0000x reference)
import functools

import jax
import jax.numpy as jnp
from jax import lax
from jax.experimental import pallas as pl
from jax.experimental.pallas import tpu as pltpu
from jax.experimental.pallas import tpu_sc as plsc

N = 10000
E = 160000
F_IN = 400
F_HID = 128
NC = 2
NS = 16
DH = F_HID // NC

KD = 40
CHD = E // (NC * NS * KD)
KR = 80
CHR = E // (NS * KR)

RPT = N // NS

_mesh = plsc.VectorSubcoreMesh(core_axis_name="c", subcore_axis_name="s")


@functools.partial(
    pl.kernel,
    out_type=jax.ShapeDtypeStruct((NC, N, 16), jnp.float32),
    mesh=_mesh,
    scratch_types=[
        pltpu.VMEM((CHD, KD), jnp.int32),
        pltpu.VMEM((KD, 16), jnp.float32),
        pltpu.VMEM_SHARED((N, 16), jnp.float32),
    ],
)
def _deg_call(dst_hbm, zeros_hbm, out_hbm, idx_v, ones_v, acc_sh):
    cid = lax.axis_index("c")
    sid = lax.axis_index("s")
    pltpu.sync_copy(zeros_hbm.at[pl.ds(sid * RPT, RPT)],
                    acc_sh.at[pl.ds(sid * RPT, RPT)])
    onerow = jnp.where(lax.iota(jnp.int32, 16) == 0, 1.0, 0.0)
    for r in range(KD):
        ones_v[r, :] = onerow
    pltpu.sync_copy(dst_hbm.at[cid, sid], idx_v)
    plsc.subcore_barrier()

    def body(j, carry):
        pltpu.sync_copy(ones_v, acc_sh.at[idx_v.at[j]], add=True)
        return carry

    lax.fori_loop(0, CHD, body, 0)
    plsc.subcore_barrier()
    pltpu.sync_copy(acc_sh.at[pl.ds(sid * RPT, RPT)],
                    out_hbm.at[cid, pl.ds(sid * RPT, RPT)])


@functools.partial(
    pl.kernel,
    out_type=jax.ShapeDtypeStruct((NC, N, DH), jnp.float32),
    mesh=_mesh,
    scratch_types=[
        pltpu.VMEM((CHR, KR), jnp.int32),
        pltpu.VMEM((CHR, KR), jnp.int32),
        pltpu.VMEM((KR, DH), jnp.float32),
        pltpu.VMEM_SHARED((N, DH), jnp.float32),
        pltpu.SemaphoreType.DMA,
    ],
)
def _scatter_call(t_hbm, src_hbm, dst_hbm, zeros_hbm, out_hbm,
                  src_v, dst_v, rows_v, acc_sh, sem):
    cid = lax.axis_index("c")
    sid = lax.axis_index("s")
    pltpu.sync_copy(zeros_hbm.at[pl.ds(sid * RPT, RPT)],
                    acc_sh.at[pl.ds(sid * RPT, RPT)])
    pltpu.sync_copy(src_hbm.at[sid], src_v)
    pltpu.sync_copy(dst_hbm.at[sid], dst_v)
    plsc.subcore_barrier()

    def body(j, carry):
        pltpu.async_copy(t_hbm.at[cid].at[src_v.at[j]], rows_v, sem).wait()
        pltpu.sync_copy(rows_v, acc_sh.at[dst_v.at[j]], add=True)
        return carry

    lax.fori_loop(0, CHR, body, 0)
    plsc.subcore_barrier()
    pltpu.sync_copy(acc_sh.at[pl.ds(sid * RPT, RPT)],
                    out_hbm.at[cid, pl.ds(sid * RPT, RPT)])


BN = 1000


def _dinv_of(degp_ref):
    deg = degp_ref[0, :, 0] + degp_ref[1, :, 0] + 1.0
    return lax.rsqrt(deg)[:, None]


def _mm1_body(x_ref, w_ref, degp_ref, o_ref):
    dinv = _dinv_of(degp_ref)
    t = jnp.dot(x_ref[:], w_ref[:], preferred_element_type=jnp.float32) * dinv
    o_ref[0] = t[:, :DH]
    o_ref[1] = t[:, DH:]


def _mid_body(acc_ref, t_ref, degp_ref, b_ref, w_ref, o_ref):
    dinv = _dinv_of(degp_ref)
    s = jnp.concatenate([acc_ref[0] + t_ref[0], acc_ref[1] + t_ref[1]], axis=1)
    h = jnp.maximum(dinv * s + b_ref[:], 0.0)
    t2 = jnp.dot(h, w_ref[:], preferred_element_type=jnp.float32) * dinv
    o_ref[0] = t2[:, :DH]
    o_ref[1] = t2[:, DH:]


def _final_body(acc_ref, t_ref, degp_ref, b_ref, o_ref):
    dinv = _dinv_of(degp_ref)
    s = jnp.concatenate([acc_ref[0] + t_ref[0], acc_ref[1] + t_ref[1]], axis=1)
    o_ref[:] = jnp.maximum(dinv * s + b_ref[:], 0.0)


_split_spec = pl.BlockSpec((NC, BN, DH), lambda i: (0, i, 0))
_degp_spec = pl.BlockSpec((NC, BN, 16), lambda i: (0, i, 0))
_bias_spec = pl.BlockSpec((1, F_HID), lambda i: (0, 0))

_mm1 = pl.pallas_call(
    _mm1_body,
    grid=(N // BN,),
    in_specs=[
        pl.BlockSpec((BN, F_IN), lambda i: (i, 0)),
        pl.BlockSpec((F_IN, F_HID), lambda i: (0, 0)),
        _degp_spec,
    ],
    out_specs=_split_spec,
    out_shape=jax.ShapeDtypeStruct((NC, N, DH), jnp.float32),
)

_mid = pl.pallas_call(
    _mid_body,
    grid=(N // BN,),
    in_specs=[
        _split_spec,
        _split_spec,
        _degp_spec,
        _bias_spec,
        pl.BlockSpec((F_HID, F_HID), lambda i: (0, 0)),
    ],
    out_specs=_split_spec,
    out_shape=jax.ShapeDtypeStruct((NC, N, DH), jnp.float32),
)

_final = pl.pallas_call(
    _final_body,
    grid=(N // BN,),
    in_specs=[_split_spec, _split_spec, _degp_spec, _bias_spec],
    out_specs=pl.BlockSpec((BN, F_HID), lambda i: (i, 0)),
    out_shape=jax.ShapeDtypeStruct((N, F_HID), jnp.float32),
)


def kernel(x, edge_index, batch, W1, b1, W2, b2):
    src = edge_index[0]
    dst = edge_index[1]
    dst_deg = dst.reshape(NC, NS, CHD, KD)
    src_r = src.reshape(NS, CHR, KR)
    dst_r = dst.reshape(NS, CHR, KR)
    zeros16 = jnp.zeros((N, 16), jnp.float32)
    zeros_dh = jnp.zeros((N, DH), jnp.float32)
    b1r = b1.reshape(1, F_HID)
    b2r = b2.reshape(1, F_HID)

    degp = _deg_call(dst_deg, zeros16)
    t1 = _mm1(x, W1, degp)
    acc1 = _scatter_call(t1, src_r, dst_r, zeros_dh)
    t2 = _mid(acc1, t1, degp, b1r, W2)
    acc2 = _scatter_call(t2, src_r, dst_r, zeros_dh)
    return _final(acc2, t2, degp, b2r)


# speedup vs baseline: 12.3745x; 12.3745x over previous
import functools

import jax
import jax.numpy as jnp
from jax import lax
from jax.experimental import pallas as pl
from jax.experimental.pallas import tpu as pltpu
from jax.experimental.pallas import tpu_sc as plsc

N = 10000
E = 160000
F_IN = 400
F_HID = 128
NC = 2
NS = 16
DH = F_HID // NC

KD = 40
CHD = E // (NC * NS * KD)
KR = 80
CHR = E // (NS * KR)

RPT = N // NS
ZR = 125

_mesh = plsc.VectorSubcoreMesh(core_axis_name="c", subcore_axis_name="s")
_sc_params = pltpu.CompilerParams(use_tc_tiling_on_sc=False)


def _zero_fill(zbuf, nrows, width):
    zv = jnp.zeros((16,), jnp.float32)

    def zrow(r, carry):
        for c in range(width // 16):
            zbuf[r, pl.ds(c * 16, 16)] = zv
        return carry

    lax.fori_loop(0, nrows, zrow, 0)


@functools.partial(
    pl.kernel,
    out_type=jax.ShapeDtypeStruct((NC, NS, RPT, 16), jnp.float32),
    mesh=_mesh,
    scratch_types=[
        pltpu.VMEM((CHD, KD), jnp.int32),
        pltpu.VMEM((KD, 16), jnp.float32),
        pltpu.VMEM((ZR, 16), jnp.float32),
        pltpu.VMEM_SHARED((N, 16), jnp.float32),
    ],
    compiler_params=_sc_params,
)
def _deg_call(dst_hbm, out_hbm, idx_v, ones_v, zbuf, acc_sh):
    cid = lax.axis_index("c")
    sid = lax.axis_index("s")
    _zero_fill(zbuf, ZR, 16)
    for q in range(RPT // ZR):
        pltpu.sync_copy(zbuf, acc_sh.at[pl.ds(sid * RPT + q * ZR, ZR)])
    onerow = jnp.where(lax.iota(jnp.int32, 16) == 0, 1.0, 0.0)
    for r in range(KD):
        ones_v[r, :] = onerow
    pltpu.sync_copy(dst_hbm.at[cid, sid], idx_v)
    plsc.subcore_barrier()

    def body(j, carry):
        pltpu.sync_copy(ones_v, acc_sh.at[idx_v.at[j]], add=True)
        return carry

    lax.fori_loop(0, CHD, body, 0)
    plsc.subcore_barrier()
    pltpu.sync_copy(acc_sh.at[pl.ds(sid * RPT, RPT)], out_hbm.at[cid, sid])


@functools.partial(
    pl.kernel,
    out_type=jax.ShapeDtypeStruct((NC, NS, RPT, DH), jnp.float32),
    mesh=_mesh,
    scratch_types=[
        pltpu.VMEM((CHR, KR), jnp.int32),
        pltpu.VMEM((CHR, KR), jnp.int32),
        pltpu.VMEM((KR, DH), jnp.float32),
        pltpu.VMEM((ZR, DH), jnp.float32),
        pltpu.VMEM_SHARED((N, DH), jnp.float32),
        pltpu.SemaphoreType.DMA,
    ],
    compiler_params=_sc_params,
)
def _scatter_call(t2_hbm, idx2_hbm, dst_hbm, out_hbm,
                  src_v, dst_v, rows_v, zbuf, acc_sh, sem):
    cid = lax.axis_index("c")
    sid = lax.axis_index("s")
    _zero_fill(zbuf, ZR, DH)
    for q in range(RPT // ZR):
        pltpu.sync_copy(zbuf, acc_sh.at[pl.ds(sid * RPT + q * ZR, ZR)])
    pltpu.sync_copy(idx2_hbm.at[cid, sid], src_v)
    pltpu.sync_copy(dst_hbm.at[sid], dst_v)
    plsc.subcore_barrier()

    def body(j, carry):
        pltpu.async_copy(t2_hbm.at[src_v.at[j]], rows_v, sem).wait()
        pltpu.sync_copy(rows_v, acc_sh.at[dst_v.at[j]], add=True)
        return carry

    lax.fori_loop(0, CHR, body, 0)
    plsc.subcore_barrier()
    pltpu.sync_copy(acc_sh.at[pl.ds(sid * RPT, RPT)], out_hbm.at[cid, sid])


BN = 1000


def _dinv_of(degp_ref):
    deg = degp_ref[0, :, 0] + degp_ref[1, :, 0] + 1.0
    return lax.rsqrt(deg)[:, None]


def _assemble(acc_ref, t_ref):
    return jnp.concatenate([acc_ref[0], acc_ref[1]], axis=1) + t_ref[:]


def _mm1_body(x_ref, w_ref, degp_ref, o_ref):
    dinv = _dinv_of(degp_ref)
    o_ref[:] = jnp.dot(x_ref[:], w_ref[:],
                       preferred_element_type=jnp.float32) * dinv


def _mid_body(acc_ref, t_ref, degp_ref, b_ref, w_ref, o_ref):
    dinv = _dinv_of(degp_ref)
    h = jnp.maximum(dinv * _assemble(acc_ref, t_ref) + b_ref[:], 0.0)
    o_ref[:] = jnp.dot(h, w_ref[:], preferred_element_type=jnp.float32) * dinv


def _final_body(acc_ref, t_ref, degp_ref, b_ref, o_ref):
    dinv = _dinv_of(degp_ref)
    o_ref[:] = jnp.maximum(dinv * _assemble(acc_ref, t_ref) + b_ref[:], 0.0)


_t_spec = pl.BlockSpec((BN, F_HID), lambda i: (i, 0))
_acc_spec = pl.BlockSpec((NC, BN, DH), lambda i: (0, i, 0))
_degp_spec = pl.BlockSpec((NC, BN, 16), lambda i: (0, i, 0))
_bias_spec = pl.BlockSpec((1, F_HID), lambda i: (0, 0))

_mm1 = pl.pallas_call(
    _mm1_body,
    grid=(N // BN,),
    in_specs=[
        pl.BlockSpec((BN, F_IN), lambda i: (i, 0)),
        pl.BlockSpec((F_IN, F_HID), lambda i: (0, 0)),
        _degp_spec,
    ],
    out_specs=_t_spec,
    out_shape=jax.ShapeDtypeStruct((N, F_HID), jnp.float32),
)

_mid = pl.pallas_call(
    _mid_body,
    grid=(N // BN,),
    in_specs=[
        _acc_spec,
        _t_spec,
        _degp_spec,
        _bias_spec,
        pl.BlockSpec((F_HID, F_HID), lambda i: (0, 0)),
    ],
    out_specs=_t_spec,
    out_shape=jax.ShapeDtypeStruct((N, F_HID), jnp.float32),
)

_final = pl.pallas_call(
    _final_body,
    grid=(N // BN,),
    in_specs=[_acc_spec, _t_spec, _degp_spec, _bias_spec],
    out_specs=_t_spec,
    out_shape=jax.ShapeDtypeStruct((N, F_HID), jnp.float32),
)


def kernel(x, edge_index, batch, W1, b1, W2, b2):
    src = edge_index[0]
    dst = edge_index[1]
    dst_deg = dst.reshape(NC, NS, CHD, KD)
    src2 = 2 * src.reshape(NS, CHR, KR)
    idx2 = jnp.stack([src2, src2 + 1])
    dst_r = dst.reshape(NS, CHR, KR)
    b1r = b1.reshape(1, F_HID)
    b2r = b2.reshape(1, F_HID)

    degp = _deg_call(dst_deg).reshape(NC, N, 16)
    t1 = _mm1(x, W1, degp)
    acc1 = _scatter_call(t1.reshape(2 * N, DH), idx2, dst_r)
    acc1 = acc1.reshape(NC, N, DH)
    t2 = _mid(acc1, t1, degp, b1r, W2)
    acc2 = _scatter_call(t2.reshape(2 * N, DH), idx2, dst_r)
    acc2 = acc2.reshape(NC, N, DH)
    return _final(acc2, t2, degp, b2r)


# speedup vs baseline: 18.0065x; 1.4551x over previous
import functools

import jax
import jax.numpy as jnp
from jax import lax
from jax.experimental import pallas as pl
from jax.experimental.pallas import tpu as pltpu
from jax.experimental.pallas import tpu_sc as plsc

N = 10000
E = 160000
F_IN = 400
F_HID = 128
NC = 2
NS = 16
DH = F_HID // NC

KD = 40
CHD = E // (NC * NS * KD)
KR = 80
CHR = E // (NS * KR)
GRP = 5

RPT = N // NS
ZR = 125

_mesh = plsc.VectorSubcoreMesh(core_axis_name="c", subcore_axis_name="s")
_sc_params = pltpu.CompilerParams(use_tc_tiling_on_sc=False)


def _zero_fill(zbuf, nrows, width):
    zv = jnp.zeros((16,), jnp.float32)

    def zrow(r, carry):
        for c in range(width // 16):
            zbuf[r, pl.ds(c * 16, 16)] = zv
        return carry

    lax.fori_loop(0, nrows, zrow, 0)


@functools.partial(
    pl.kernel,
    out_type=jax.ShapeDtypeStruct((NC, NS, RPT, 16), jnp.float32),
    mesh=_mesh,
    scratch_types=[
        pltpu.VMEM((CHD, KD), jnp.int32),
        pltpu.VMEM((KD, 16), jnp.float32),
        pltpu.VMEM((ZR, 16), jnp.float32),
        pltpu.VMEM_SHARED((N, 16), jnp.float32),
    ],
    compiler_params=_sc_params,
)
def _deg_call(dst_hbm, out_hbm, idx_v, ones_v, zbuf, acc_sh):
    cid = lax.axis_index("c")
    sid = lax.axis_index("s")
    _zero_fill(zbuf, ZR, 16)
    for q in range(RPT // ZR):
        pltpu.sync_copy(zbuf, acc_sh.at[pl.ds(sid * RPT + q * ZR, ZR)])
    onerow = jnp.where(lax.iota(jnp.int32, 16) == 0, 1.0, 0.0)
    for r in range(KD):
        ones_v[r, :] = onerow
    pltpu.sync_copy(dst_hbm.at[cid, sid], idx_v)
    plsc.subcore_barrier()

    def body(j, carry):
        pltpu.sync_copy(ones_v, acc_sh.at[idx_v.at[j]], add=True)
        return carry

    lax.fori_loop(0, CHD, body, 0)
    plsc.subcore_barrier()
    pltpu.sync_copy(acc_sh.at[pl.ds(sid * RPT, RPT)], out_hbm.at[cid, sid])


@functools.partial(
    pl.kernel,
    out_type=jax.ShapeDtypeStruct((NC, NS, RPT, DH), jnp.float32),
    mesh=_mesh,
    scratch_types=[
        pltpu.VMEM((CHR, KR), jnp.int32),
        pltpu.VMEM((CHR, KR), jnp.int32),
        pltpu.VMEM((GRP, KR, DH), jnp.float32),
        pltpu.VMEM((ZR, DH), jnp.float32),
        pltpu.VMEM_SHARED((N, DH), jnp.float32),
    ] + [pltpu.SemaphoreType.DMA] * GRP,
    compiler_params=_sc_params,
)
def _scatter_call(t2_hbm, idx2_hbm, dst_hbm, out_hbm,
                  src_v, dst_v, rows_v, zbuf, acc_sh, *sems):
    cid = lax.axis_index("c")
    sid = lax.axis_index("s")
    _zero_fill(zbuf, ZR, DH)
    for q in range(RPT // ZR):
        pltpu.sync_copy(zbuf, acc_sh.at[pl.ds(sid * RPT + q * ZR, ZR)])
    pltpu.sync_copy(idx2_hbm.at[cid, sid], src_v)
    pltpu.sync_copy(dst_hbm.at[sid], dst_v)
    plsc.subcore_barrier()

    def body(p, carry):
        j = GRP * p
        copies = [
            pltpu.async_copy(t2_hbm.at[src_v.at[j + i]], rows_v.at[i], sems[i])
            for i in range(GRP)
        ]
        for i in range(GRP):
            copies[i].wait()
            pltpu.sync_copy(rows_v.at[i], acc_sh.at[dst_v.at[j + i]], add=True)
        return carry

    lax.fori_loop(0, CHR // GRP, body, 0)
    plsc.subcore_barrier()
    pltpu.sync_copy(acc_sh.at[pl.ds(sid * RPT, RPT)], out_hbm.at[cid, sid])


BN = 1000


def _dinv_of(degp_ref):
    deg = degp_ref[0, :, 0] + degp_ref[1, :, 0] + 1.0
    return lax.rsqrt(deg)[:, None]


def _assemble(acc_ref, t_ref):
    return jnp.concatenate([acc_ref[0], acc_ref[1]], axis=1) + t_ref[:]


def _mm1_body(x_ref, w_ref, degp_ref, o_ref):
    dinv = _dinv_of(degp_ref)
    o_ref[:] = jnp.dot(x_ref[:], w_ref[:],
                       preferred_element_type=jnp.float32) * dinv


def _mid_body(acc_ref, t_ref, degp_ref, b_ref, w_ref, o_ref):
    dinv = _dinv_of(degp_ref)
    h = jnp.maximum(dinv * _assemble(acc_ref, t_ref) + b_ref[:], 0.0)
    o_ref[:] = jnp.dot(h, w_ref[:], preferred_element_type=jnp.float32) * dinv


def _final_body(acc_ref, t_ref, degp_ref, b_ref, o_ref):
    dinv = _dinv_of(degp_ref)
    o_ref[:] = jnp.maximum(dinv * _assemble(acc_ref, t_ref) + b_ref[:], 0.0)


_t_spec = pl.BlockSpec((BN, F_HID), lambda i: (i, 0))
_acc_spec = pl.BlockSpec((NC, BN, DH), lambda i: (0, i, 0))
_degp_spec = pl.BlockSpec((NC, BN, 16), lambda i: (0, i, 0))
_bias_spec = pl.BlockSpec((1, F_HID), lambda i: (0, 0))

_mm1 = pl.pallas_call(
    _mm1_body,
    grid=(N // BN,),
    in_specs=[
        pl.BlockSpec((BN, F_IN), lambda i: (i, 0)),
        pl.BlockSpec((F_IN, F_HID), lambda i: (0, 0)),
        _degp_spec,
    ],
    out_specs=_t_spec,
    out_shape=jax.ShapeDtypeStruct((N, F_HID), jnp.float32),
)

_mid = pl.pallas_call(
    _mid_body,
    grid=(N // BN,),
    in_specs=[
        _acc_spec,
        _t_spec,
        _degp_spec,
        _bias_spec,
        pl.BlockSpec((F_HID, F_HID), lambda i: (0, 0)),
    ],
    out_specs=_t_spec,
    out_shape=jax.ShapeDtypeStruct((N, F_HID), jnp.float32),
)

_final = pl.pallas_call(
    _final_body,
    grid=(N // BN,),
    in_specs=[_acc_spec, _t_spec, _degp_spec, _bias_spec],
    out_specs=_t_spec,
    out_shape=jax.ShapeDtypeStruct((N, F_HID), jnp.float32),
)


def kernel(x, edge_index, batch, W1, b1, W2, b2):
    src = edge_index[0]
    dst = edge_index[1]
    dst_deg = dst.reshape(NC, NS, CHD, KD)
    src2 = 2 * src.reshape(NS, CHR, KR)
    idx2 = jnp.stack([src2, src2 + 1])
    dst_r = dst.reshape(NS, CHR, KR)
    b1r = b1.reshape(1, F_HID)
    b2r = b2.reshape(1, F_HID)

    degp = _deg_call(dst_deg).reshape(NC, N, 16)
    t1 = _mm1(x, W1, degp)
    acc1 = _scatter_call(t1.reshape(2 * N, DH), idx2, dst_r)
    acc1 = acc1.reshape(NC, N, DH)
    t2 = _mid(acc1, t1, degp, b1r, W2)
    acc2 = _scatter_call(t2.reshape(2 * N, DH), idx2, dst_r)
    acc2 = acc2.reshape(NC, N, DH)
    return _final(acc2, t2, degp, b2r)
